# baseline (speedup 1.0000x reference)
import functools

import jax
import jax.numpy as jnp
from jax import lax
from jax.experimental import pallas as pl
from jax.experimental.pallas import tpu as pltpu
from jax.experimental.pallas import tpu_sc as plsc

NC = 2
NS = 16
NW = NC * NS
CHUNK = 120
GROUPS = 64

_mesh = lambda: plsc.VectorSubcoreMesh(core_axis_name="c", subcore_axis_name="s")


DCHUNK = 128


def _make_deg(n_pad, nchunk):
    rpt = n_pad // NS

    @functools.partial(
        pl.kernel,
        out_type=jax.ShapeDtypeStruct((NC, n_pad), jnp.float32),
        mesh=_mesh(),
        scratch_types=[
            pltpu.VMEM((nchunk, DCHUNK), jnp.int32),
            pltpu.VMEM((DCHUNK,), jnp.float32),
            pltpu.VMEM_SHARED((n_pad,), jnp.float32),
        ],
    )
    def deg_kernel(dst_hbm, ones_hbm, z_hbm, out_hbm, dstv, onesv, acc):
        c = lax.axis_index("c")
        s = lax.axis_index("s")
        wid = c * NS + s
        pltpu.sync_copy(z_hbm.at[wid], acc.at[pl.ds(s * rpt, rpt)])
        pltpu.sync_copy(dst_hbm.at[wid], dstv)
        pltpu.sync_copy(ones_hbm.at[wid], onesv)
        plsc.subcore_barrier()

        def body(j, carry):
            pltpu.sync_copy(onesv, acc.at[dstv.at[j]], add=True)
            return carry

        lax.fori_loop(0, nchunk, body, 0)
        plsc.subcore_barrier()
        pltpu.sync_copy(acc.at[pl.ds(s * rpt, rpt)],
                        out_hbm.at[c, pl.ds(s * rpt, rpt)])

    return deg_kernel


def _make_scatter(n_pad, hw, nchunk, n_rows):
    rpt = n_pad // NS

    @functools.partial(
        pl.kernel,
        out_type=jax.ShapeDtypeStruct((NC, n_pad, hw), jnp.float32),
        mesh=_mesh(),
        scratch_types=[pltpu.VMEM((1, CHUNK), jnp.int32)] * 12 + [
            pltpu.VMEM((CHUNK, hw), jnp.float32),
            pltpu.VMEM((CHUNK, hw), jnp.float32),
            pltpu.VMEM((CHUNK, hw), jnp.float32),
            pltpu.VMEM_SHARED((n_pad, hw), jnp.float32),
        ] + [pltpu.SemaphoreType.DMA] * 18,
    )
    def scat_kernel(h_hbm, src_hbm, dst_hbm, z_hbm, out_hbm, *refs):
        srcx = refs[0:6]
        dstx = refs[6:12]
        rows0, rows1, rows2, acc = refs[12:16]
        sems = refs[16:]
        semsi = sems[0:6]
        semdi = sems[6:12]
        semg = sems[12:15]
        semss = sems[15:18]
        rows = (rows0, rows1, rows2)
        c = lax.axis_index("c")
        s = lax.axis_index("s")
        wid = c * NS + s
        pltpu.sync_copy(z_hbm.at[wid], acc.at[pl.ds(s * rpt, rpt)])
        plsc.subcore_barrier()

        for u in range(6):
            pltpu.async_copy(src_hbm.at[wid, u], srcx[u], semsi[u])
            pltpu.async_copy(dst_hbm.at[wid, u], dstx[u], semdi[u])
        for t in range(2):
            pltpu.make_async_copy(src_hbm.at[wid, t], srcx[t],
                                  semsi[t]).wait()
            pltpu.async_copy(h_hbm.at[srcx[t].at[0]], rows[t], semg[t])

        def body(k, carry):
            for off in range(6):
                j = 6 * k + off
                t = off % 3
                u = off
                t2 = (off + 2) % 3
                u2 = (off + 2) % 6
                up = (off + 5) % 6
                pltpu.make_async_copy(h_hbm.at[srcx[u].at[0]], rows[t],
                                      semg[t]).wait()
                pltpu.make_async_copy(dst_hbm.at[wid, j], dstx[u],
                                      semdi[u]).wait()
                pltpu.async_copy(rows[t], acc.at[dstx[u].at[0]], semss[t],
                                 add=True)

                @pl.when(j >= 1)
                def _free_and_refetch():
                    pltpu.make_async_copy(rows[t2], acc.at[dstx[up].at[0]],
                                          semss[t2]).wait()

                    @pl.when(j + 5 < nchunk)
                    def _refetch_idx():
                        pltpu.async_copy(src_hbm.at[wid, j + 5],
                                         srcx[up], semsi[up])
                        pltpu.async_copy(dst_hbm.at[wid, j + 5],
                                         dstx[up], semdi[up])

                @pl.when(j + 2 < nchunk)
                def _launch_gather():
                    pltpu.make_async_copy(src_hbm.at[wid, j + 2],
                                          srcx[u2], semsi[u2]).wait()
                    pltpu.async_copy(h_hbm.at[srcx[u2].at[0]], rows[t2],
                                     semg[t2])

            return carry

        lax.fori_loop(0, nchunk // 6, body, 0)
        pltpu.make_async_copy(rows[2], acc.at[dstx[5].at[0]], semss[2]).wait()
        plsc.subcore_barrier()
        pltpu.sync_copy(acc.at[pl.ds(s * rpt, rpt)],
                        out_hbm.at[c, pl.ds(s * rpt, rpt)])

    return scat_kernel


def _kmm_body(x_ref, w_ref, u_ref):
    u_ref[...] = x_ref[...] @ w_ref[...]


def _ka_body(deg_ref, u_ref, dinv_ref, g_ref):
    n = u_ref.shape[0]
    deg = deg_ref[0] + deg_ref[1] + 1.0
    dinv = lax.rsqrt(deg)[:, None]
    dinv_ref[...] = dinv
    g_ref[...] = u_ref[...] * dinv[:n]


def _kmid_body(p_ref, g_ref, dinv_ref, b_ref, w_ref, gout_ref):
    n = g_ref.shape[0]
    dinv = dinv_ref[...][:n]
    agg = p_ref[0, :n, :] + p_ref[1, :n, :] + g_ref[...]
    h = jnp.maximum(agg * dinv + b_ref[...][None, :], 0.0)
    u = h @ w_ref[...]
    pad = gout_ref.shape[1] - u.shape[1]
    if pad:
        u = jnp.concatenate([u, jnp.zeros((n, pad), u.dtype)], axis=1)
    gout_ref[...] = u * dinv


def _kd_body(p_ref, g_ref, dinv_ref, b_ref, batch_ref, out_ref):
    n = g_ref.shape[0]
    c = b_ref.shape[0]
    g = out_ref.shape[0]
    dinv = dinv_ref[...][:n]
    agg = p_ref[0, :n, :] + p_ref[1, :n, :] + g_ref[...]
    h = agg[:, :c] * dinv + b_ref[...][None, :]
    bt = batch_ref[...]
    oh = (bt[:, None] == lax.broadcasted_iota(jnp.int32, (n, g), 1))
    oh = oh.astype(jnp.float32)
    sums = lax.dot_general(oh, h, (((0,), (0,)), ((), ())))
    cnt = jnp.sum(oh, axis=0)[:, None]
    pooled = sums / jnp.maximum(cnt, 1.0)
    m = jnp.max(pooled, axis=1, keepdims=True)
    e = jnp.exp(pooled - m)
    out_ref[...] = e / jnp.sum(e, axis=1, keepdims=True)


def kernel(x, edge_index, batch, W1, b1, W2, b2, W3, b3):
    n, f_in = x.shape
    h_dim = W1.shape[1]
    c_dim = W3.shape[1]
    cp = 128
    e = edge_index.shape[1]

    epw = -(-e // NW)
    nchunk = -(-epw // CHUNK)
    nchunk = -(-nchunk // 6) * 6
    e_pad = NW * nchunk * CHUNK
    rpt = -(-(n + NW) // NS)
    rpt = -(-rpt // 128) * 128
    n_pad = NS * rpt

    src = edge_index[0]
    dst = edge_index[1]
    pad = e_pad - e
    pidx = jnp.arange(pad, dtype=jnp.int32)
    src_p = jnp.concatenate([src, pidx % n])
    dst_p = jnp.concatenate([dst, n + pidx % (n_pad - n)])
    src_r = src_p.reshape(NW, nchunk, CHUNK)
    dst_r = dst_p.reshape(NW, nchunk, CHUNK)

    nchunk_d = -(-epw // DCHUNK)
    nchunk_d += nchunk_d % 2
    e_pad_d = NW * nchunk_d * DCHUNK
    pad_d = e_pad_d - e
    pidx_d = jnp.arange(pad_d, dtype=jnp.int32)
    dst_rd = jnp.concatenate([dst, n + pidx_d % (n_pad - n)])
    dst_rd = dst_rd.reshape(NW, nchunk_d, DCHUNK)

    ones_col = jnp.ones((NW, DCHUNK), jnp.float32)
    z_col = jnp.zeros((NW, rpt), jnp.float32)
    z_h = jnp.zeros((NW, rpt, h_dim), jnp.float32)

    deg_fn = _make_deg(n_pad, nchunk_d)
    scat_h = _make_scatter(n_pad, h_dim, nchunk, n)
    scat_c = scat_h if cp == h_dim else _make_scatter(n_pad, cp, nchunk, n)

    degp = deg_fn(dst_rd, ones_col, z_col)

    u1 = pl.pallas_call(
        _kmm_body,
        out_shape=jax.ShapeDtypeStruct((n, h_dim), jnp.float32),
    )(x, W1)

    dinv, g1 = pl.pallas_call(
        _ka_body,
        out_shape=(jax.ShapeDtypeStruct((n_pad, 1), jnp.float32),
                   jax.ShapeDtypeStruct((n, h_dim), jnp.float32)),
    )(degp, u1)

    src_r4 = src_r.reshape(NW, nchunk, 1, CHUNK)
    dst_r4 = dst_r.reshape(NW, nchunk, 1, CHUNK)
    p1 = scat_h(g1, src_r4, dst_r4, z_h)

    g2 = pl.pallas_call(
        _kmid_body,
        out_shape=jax.ShapeDtypeStruct((n, h_dim), jnp.float32),
    )(p1, g1, dinv, b1, W2)

    p2 = scat_h(g2, src_r4, dst_r4, z_h)

    g3 = pl.pallas_call(
        _kmid_body,
        out_shape=jax.ShapeDtypeStruct((n, cp), jnp.float32),
    )(p2, g2, dinv, b2, W3)

    p3 = scat_c(g3, src_r4, dst_r4, z_h)

    out = pl.pallas_call(
        _kd_body,
        out_shape=jax.ShapeDtypeStruct((GROUPS, c_dim), jnp.float32),
    )(p3, g3, dinv, b3, batch)
    return out


# speedup vs baseline: 1.0826x; 1.0826x over previous
import functools

import jax
import jax.numpy as jnp
from jax import lax
from jax.experimental import pallas as pl
from jax.experimental.pallas import tpu as pltpu
from jax.experimental.pallas import tpu_sc as plsc

NC = 2
NS = 16
NW = NC * NS
CHUNK = 120
GROUPS = 64

_mesh = lambda: plsc.VectorSubcoreMesh(core_axis_name="c", subcore_axis_name="s")


DCHUNK = 128


def _make_deg(n_pad, nchunk):
    rpt = n_pad // NS

    @functools.partial(
        pl.kernel,
        out_type=jax.ShapeDtypeStruct((NC, n_pad), jnp.float32),
        mesh=_mesh(),
        scratch_types=[
            pltpu.VMEM((nchunk, DCHUNK), jnp.int32),
            pltpu.VMEM((DCHUNK,), jnp.float32),
            pltpu.VMEM_SHARED((n_pad,), jnp.float32),
        ],
    )
    def deg_kernel(dst_hbm, ones_hbm, z_hbm, out_hbm, dstv, onesv, acc):
        c = lax.axis_index("c")
        s = lax.axis_index("s")
        wid = c * NS + s
        pltpu.sync_copy(z_hbm.at[wid], acc.at[pl.ds(s * rpt, rpt)])
        pltpu.sync_copy(dst_hbm.at[wid], dstv)
        pltpu.sync_copy(ones_hbm.at[wid], onesv)
        plsc.subcore_barrier()

        def body(j, carry):
            pltpu.sync_copy(onesv, acc.at[dstv.at[j]], add=True)
            return carry

        lax.fori_loop(0, nchunk, body, 0)
        plsc.subcore_barrier()
        pltpu.sync_copy(acc.at[pl.ds(s * rpt, rpt)],
                        out_hbm.at[c, pl.ds(s * rpt, rpt)])

    return deg_kernel


def _make_scatter(n_pad, hw, nchunk):
    rpt = n_pad // NS

    @functools.partial(
        pl.kernel,
        out_type=jax.ShapeDtypeStruct((NC, n_pad, hw), jnp.float32),
        mesh=_mesh(),
        scratch_types=[pltpu.VMEM((1, CHUNK), jnp.int32)] * 12 + [
            pltpu.VMEM((CHUNK, hw), jnp.float32),
            pltpu.VMEM((CHUNK, hw), jnp.float32),
            pltpu.VMEM((CHUNK, hw), jnp.float32),
            pltpu.VMEM_SHARED((n_pad, hw), jnp.float32),
        ] + [pltpu.SemaphoreType.DMA] * 18,
    )
    def scat_kernel(h_hbm, src_hbm, dst_hbm, z_hbm, out_hbm, *refs):
        srcx = refs[0:6]
        dstx = refs[6:12]
        rows0, rows1, rows2, acc = refs[12:16]
        sems = refs[16:]
        semsi = sems[0:6]
        semdi = sems[6:12]
        semg = sems[12:15]
        semss = sems[15:18]
        rows = (rows0, rows1, rows2)
        c = lax.axis_index("c")
        s = lax.axis_index("s")
        wid = c * NS + s
        pltpu.sync_copy(z_hbm.at[wid], acc.at[pl.ds(s * rpt, rpt)])
        plsc.subcore_barrier()

        for u in range(6):
            pltpu.async_copy(src_hbm.at[wid, u], srcx[u], semsi[u])
            pltpu.async_copy(dst_hbm.at[wid, u], dstx[u], semdi[u])
        for t in range(2):
            pltpu.make_async_copy(src_hbm.at[wid, t], srcx[t],
                                  semsi[t]).wait()
            pltpu.async_copy(h_hbm.at[srcx[t].at[0]], rows[t], semg[t])

        def body(k, carry):
            for off in range(6):
                j = 6 * k + off
                t = off % 3
                u = off
                t2 = (off + 2) % 3
                u2 = (off + 2) % 6
                up = (off + 5) % 6
                pltpu.make_async_copy(h_hbm.at[srcx[u].at[0]], rows[t],
                                      semg[t]).wait()
                pltpu.make_async_copy(dst_hbm.at[wid, j], dstx[u],
                                      semdi[u]).wait()
                pltpu.async_copy(rows[t], acc.at[dstx[u].at[0]], semss[t],
                                 add=True)

                @pl.when(j >= 1)
                def _free_and_refetch():
                    pltpu.make_async_copy(rows[t2], acc.at[dstx[up].at[0]],
                                          semss[t2]).wait()

                    @pl.when(j + 5 < nchunk)
                    def _refetch_idx():
                        pltpu.async_copy(src_hbm.at[wid, j + 5],
                                         srcx[up], semsi[up])
                        pltpu.async_copy(dst_hbm.at[wid, j + 5],
                                         dstx[up], semdi[up])

                @pl.when(j + 2 < nchunk)
                def _launch_gather():
                    pltpu.make_async_copy(src_hbm.at[wid, j + 2],
                                          srcx[u2], semsi[u2]).wait()
                    pltpu.async_copy(h_hbm.at[srcx[u2].at[0]], rows[t2],
                                     semg[t2])

            return carry

        lax.fori_loop(0, nchunk // 6, body, 0)
        pltpu.make_async_copy(rows[2], acc.at[dstx[5].at[0]], semss[2]).wait()
        plsc.subcore_barrier()
        pltpu.sync_copy(acc.at[pl.ds(s * rpt, rpt)],
                        out_hbm.at[c, pl.ds(s * rpt, rpt)])

    return scat_kernel


def _make_scatter_narrow(n_pad, hw, nchunk):
    rpt = n_pad // NS

    @functools.partial(
        pl.kernel,
        out_type=jax.ShapeDtypeStruct((NC, n_pad, hw), jnp.float32),
        mesh=_mesh(),
        compiler_params=pltpu.CompilerParams(use_tc_tiling_on_sc=False),
        scratch_types=[
            pltpu.VMEM((nchunk, CHUNK), jnp.int32),
            pltpu.VMEM((nchunk, CHUNK), jnp.int32),
            pltpu.VMEM((CHUNK, hw), jnp.float32),
            pltpu.VMEM((CHUNK, hw), jnp.float32),
            pltpu.VMEM((CHUNK, hw), jnp.float32),
            pltpu.VMEM_SHARED((n_pad, hw), jnp.float32),
        ] + [pltpu.SemaphoreType.DMA] * 6,
    )
    def scatn_kernel(h_hbm, src_hbm, dst_hbm, z_hbm, out_hbm,
                     srcv, dstv, rows0, rows1, rows2, acc, *sems):
        semg = sems[0:3]
        semss = sems[3:6]
        rows = (rows0, rows1, rows2)
        c = lax.axis_index("c")
        s = lax.axis_index("s")
        wid = c * NS + s
        pltpu.sync_copy(z_hbm.at[wid], acc.at[pl.ds(s * rpt, rpt)])
        pltpu.sync_copy(src_hbm.at[wid], srcv)
        pltpu.sync_copy(dst_hbm.at[wid], dstv)
        plsc.subcore_barrier()

        for t in range(2):
            pltpu.async_copy(h_hbm.at[srcv.at[t]], rows[t], semg[t])

        def body(k, carry):
            for off in range(3):
                j = 3 * k + off
                t = off
                t2 = (off + 2) % 3
                pltpu.make_async_copy(h_hbm.at[srcv.at[j]], rows[t],
                                      semg[t]).wait()
                pltpu.async_copy(rows[t], acc.at[dstv.at[j]], semss[t],
                                 add=True)

                @pl.when(j >= 1)
                def _free_prev():
                    pltpu.make_async_copy(rows[t2], acc.at[dstv.at[j]],
                                          semss[t2]).wait()

                @pl.when(j + 2 < nchunk)
                def _launch_gather():
                    pltpu.async_copy(h_hbm.at[srcv.at[j + 2]], rows[t2],
                                     semg[t2])

            return carry

        lax.fori_loop(0, nchunk // 3, body, 0)
        pltpu.make_async_copy(rows[2], acc.at[dstv.at[0]], semss[2]).wait()
        plsc.subcore_barrier()
        pltpu.sync_copy(acc.at[pl.ds(s * rpt, rpt)],
                        out_hbm.at[c, pl.ds(s * rpt, rpt)])

    return scatn_kernel


def _ka_body(deg_ref, x_ref, w_ref, dinv_ref, g_ref):
    n = x_ref.shape[0]
    deg = deg_ref[0] + deg_ref[1] + 1.0
    dinv = lax.rsqrt(deg)[:, None]
    dinv_ref[...] = dinv
    g_ref[...] = (x_ref[...] @ w_ref[...]) * dinv[:n]


def _kmid_body(p_ref, g_ref, dinv_ref, b_ref, w_ref, gout_ref):
    n = g_ref.shape[0]
    dinv = dinv_ref[...][:n]
    agg = p_ref[0, :n, :] + p_ref[1, :n, :] + g_ref[...]
    h = jnp.maximum(agg * dinv + b_ref[...][None, :], 0.0)
    u = h @ w_ref[...]
    pad = gout_ref.shape[1] - u.shape[1]
    if pad:
        u = jnp.concatenate([u, jnp.zeros((n, pad), u.dtype)], axis=1)
    gout_ref[...] = u * dinv


def _kd_body(p_ref, g_ref, dinv_ref, b_ref, batch_ref, out_ref):
    n = g_ref.shape[0]
    c = b_ref.shape[0]
    g = out_ref.shape[0]
    dinv = dinv_ref[...][:n]
    agg = p_ref[0, :n, :] + p_ref[1, :n, :] + g_ref[...]
    h = agg[:, :c] * dinv + b_ref[...][None, :]
    bt = batch_ref[...]
    oh = (bt[:, None] == lax.broadcasted_iota(jnp.int32, (n, g), 1))
    oh = oh.astype(jnp.float32)
    sums = lax.dot_general(oh, h, (((0,), (0,)), ((), ())))
    cnt = jnp.sum(oh, axis=0)[:, None]
    pooled = sums / jnp.maximum(cnt, 1.0)
    m = jnp.max(pooled, axis=1, keepdims=True)
    e = jnp.exp(pooled - m)
    out_ref[...] = e / jnp.sum(e, axis=1, keepdims=True)


def kernel(x, edge_index, batch, W1, b1, W2, b2, W3, b3):
    n, f_in = x.shape
    h_dim = W1.shape[1]
    c_dim = W3.shape[1]
    cp = 64
    e = edge_index.shape[1]

    epw = -(-e // NW)
    nchunk = -(-epw // CHUNK)
    nchunk = -(-nchunk // 6) * 6
    e_pad = NW * nchunk * CHUNK
    rpt = -(-(n + NW) // NS)
    rpt = -(-rpt // 128) * 128
    n_pad = NS * rpt

    src = edge_index[0]
    dst = edge_index[1]
    pad = e_pad - e
    pidx = jnp.arange(pad, dtype=jnp.int32)
    src_p = jnp.concatenate([src, pidx % n])
    dst_p = jnp.concatenate([dst, n + pidx % (n_pad - n)])
    src_r = src_p.reshape(NW, nchunk, CHUNK)
    dst_r = dst_p.reshape(NW, nchunk, CHUNK)

    nchunk_d = -(-epw // DCHUNK)
    nchunk_d += nchunk_d % 2
    e_pad_d = NW * nchunk_d * DCHUNK
    pad_d = e_pad_d - e
    pidx_d = jnp.arange(pad_d, dtype=jnp.int32)
    dst_rd = jnp.concatenate([dst, n + pidx_d % (n_pad - n)])
    dst_rd = dst_rd.reshape(NW, nchunk_d, DCHUNK)

    ones_col = jnp.ones((NW, DCHUNK), jnp.float32)
    z_col = jnp.zeros((NW, rpt), jnp.float32)
    z_h = jnp.zeros((NW, rpt, h_dim), jnp.float32)

    deg_fn = _make_deg(n_pad, nchunk_d)
    scat_h = _make_scatter(n_pad, h_dim, nchunk)
    scat_c = scat_h if cp == h_dim else _make_scatter_narrow(n_pad, cp,
                                                             nchunk)

    degp = deg_fn(dst_rd, ones_col, z_col)

    dinv, g1 = pl.pallas_call(
        _ka_body,
        out_shape=(jax.ShapeDtypeStruct((n_pad, 1), jnp.float32),
                   jax.ShapeDtypeStruct((n, h_dim), jnp.float32)),
    )(degp, x, W1)

    src_r4 = src_r.reshape(NW, nchunk, 1, CHUNK)
    dst_r4 = dst_r.reshape(NW, nchunk, 1, CHUNK)
    p1 = scat_h(g1, src_r4, dst_r4, z_h)

    g2 = pl.pallas_call(
        _kmid_body,
        out_shape=jax.ShapeDtypeStruct((n, h_dim), jnp.float32),
    )(p1, g1, dinv, b1, W2)

    p2 = scat_h(g2, src_r4, dst_r4, z_h)

    g3 = pl.pallas_call(
        _kmid_body,
        out_shape=jax.ShapeDtypeStruct((n, cp), jnp.float32),
    )(p2, g2, dinv, b2, W3)

    z_c = jnp.zeros((NW, rpt, cp), jnp.float32)
    p3 = scat_c(g3, src_r, dst_r, z_c)

    out = pl.pallas_call(
        _kd_body,
        out_shape=jax.ShapeDtypeStruct((GROUPS, c_dim), jnp.float32),
    )(p3, g3, dinv, b3, batch)
    return out
